# baseline (speedup 1.0000x reference)
import functools
import jax
import jax.numpy as jnp
from jax import lax
from jax.experimental import pallas as pl
from jax.experimental.pallas import tpu as pltpu
from jax.experimental.pallas import tpu_sc as plsc

FRAME = 2048
HOP = 512
FV = 4096
BV = 4
OUT_LEN = FV * HOP

NSC = 2
NSUB = 16
NW = NSC * NSUB
ROWS_PER_W = FV // NW
BLK = 8
NSTEP = ROWS_PER_W // BLK
ZROW = 32


def _sc_body(x_hbm, out_hbm, xbuf, ob0, ob1, ob2, ob3, sems, osems):
    wid = lax.axis_index("s") * NSC + lax.axis_index("c")
    blk0 = wid * NSTEP
    row0 = blk0 * BLK
    obufs = [ob0, ob1, ob2, ob3]

    z = jnp.zeros((16,), jnp.float32)
    for cc in range(FRAME // 16):
        xbuf[ZROW, pl.ds(cc * 16, 16)] = z

    def prime(u):
        dst0 = pl.multiple_of((row0 + u * BLK) * HOP, 4096)
        pltpu.make_async_copy(
            obufs[u], out_hbm.at[0, pl.ds(dst0, BLK * HOP)],
            osems.at[u]).start()

    for u in range(4):
        prime(u)

    def fetch(b, blk, slot):
        blk_c = jnp.clip(blk, 0, FV // BLK - 1)
        row = pl.multiple_of(blk_c * BLK, BLK)
        return pltpu.make_async_copy(
            x_hbm.at[b, pl.ds(row, BLK), :],
            xbuf.at[pl.ds(slot * BLK, BLK)],
            sems.at[slot])

    def out_copy(b, j, u):
        dst0 = pl.multiple_of((row0 + j * BLK) * HOP, 4096)
        return pltpu.make_async_copy(
            obufs[u], out_hbm.at[b, pl.ds(dst0, BLK * HOP)], osems.at[u])

    def group(idx, carry):
        b = idx // 4
        gi = idx % 4

        @pl.when(gi == 0)
        def _():
            @pl.when(idx > 0)
            def _():
                fetch(b, blk0, 1).wait()
            fetch(b, blk0 - 1, 3).start()
            fetch(b, blk0, 0).start()
            fetch(b, blk0 + 1, 1).start()
            fetch(b, blk0 - 1, 3).wait()
            fetch(b, blk0, 0).wait()

        for u in range(4):
            j = gi * 4 + u
            fetch(b, blk0 + j + 2, (u + 2) % 4).start()
            fetch(b, blk0 + j + 1, (u + 1) % 4).wait()
            out_copy(b, j, u).wait()

            obuf = obufs[u]
            o0 = row0 + j * BLK

            @plsc.parallel_loop(0, BLK)
            def row(t):
                o = o0 + t
                for half in range(2):
                    h = o + 1 + half
                    cnt = jnp.minimum(jnp.minimum(h + 1, 4), 4099 - h)
                    recip = jnp.where(
                        cnt == 4, jnp.float32(0.25),
                        jnp.where(cnt == 3, jnp.float32(1.0 / 3.0),
                                  jnp.where(cnt == 2, jnp.float32(0.5),
                                            jnp.float32(1.0))))
                    rv = jnp.broadcast_to(recip, (16,))
                    rows = []
                    for k in range(4):
                        gr = h - k
                        valid = jnp.logical_and(gr >= 0, gr < FV)
                        rows.append(jnp.where(valid, gr & 31, ZROW))
                    for c in range(HOP // 32):
                        off = (1 - half) * 256 + 16 * c
                        v = xbuf[rows[0], pl.ds(off, 16)]
                        v = v + xbuf[rows[1], pl.ds(512 + off, 16)]
                        v = v + xbuf[rows[2], pl.ds(1024 + off, 16)]
                        v = v + xbuf[rows[3], pl.ds(1536 + off, 16)]
                        obuf[pl.ds(t * HOP + half * 256 + 16 * c, 16)] = v * rv

            out_copy(b, j, u).start()
        return carry

    lax.fori_loop(0, BV * 4, group, 0)

    fetch(BV - 1, blk0, 1).wait()
    for u in range(4):
        out_copy(BV - 1, 12 + u, u).wait()


def kernel(x):
    mesh = plsc.VectorSubcoreMesh(core_axis_name="c", subcore_axis_name="s")
    f = functools.partial(
        pl.kernel,
        mesh=mesh,
        out_type=jax.ShapeDtypeStruct((BV, OUT_LEN), jnp.float32),
        scratch_types=[
            pltpu.VMEM((ZROW + 1, FRAME), jnp.float32),
            pltpu.VMEM((BLK * HOP,), jnp.float32),
            pltpu.VMEM((BLK * HOP,), jnp.float32),
            pltpu.VMEM((BLK * HOP,), jnp.float32),
            pltpu.VMEM((BLK * HOP,), jnp.float32),
            pltpu.SemaphoreType.DMA((4,)),
            pltpu.SemaphoreType.DMA((4,)),
        ],
    )(_sc_body)
    return f(x)


# speedup vs baseline: 1.0195x; 1.0195x over previous
import functools
import jax
import jax.numpy as jnp
from jax import lax
from jax.experimental import pallas as pl
from jax.experimental.pallas import tpu as pltpu
from jax.experimental.pallas import tpu_sc as plsc

FRAME = 2048
HOP = 512
FV = 4096
BV = 4
OUT_LEN = FV * HOP

NSC = 2
NSUB = 16
NW = NSC * NSUB
ROWS_PER_W = FV // NW
BLK = 8
NSTEP = ROWS_PER_W // BLK
ZROW = 32


def _sc_body(x_hbm, out_hbm, xbuf, ob0, ob1, ob2, ob3, sems, osems):
    wid = lax.axis_index("s") * NSC + lax.axis_index("c")
    blk0 = wid * NSTEP
    row0 = blk0 * BLK
    obufs = [ob0, ob1, ob2, ob3]

    z = jnp.zeros((16,), jnp.float32)
    for cc in range(FRAME // 16):
        xbuf[ZROW, pl.ds(cc * 16, 16)] = z

    def fetch(b, blk, slot):
        blk_c = jnp.clip(blk, 0, FV // BLK - 1)
        row = pl.multiple_of(blk_c * BLK, BLK)
        return pltpu.make_async_copy(
            x_hbm.at[b, pl.ds(row, BLK), :],
            xbuf.at[pl.ds(slot * BLK, BLK)],
            sems.at[slot])

    def out_copy(b, j, u):
        dst0 = pl.multiple_of((row0 + j * BLK) * HOP, 4096)
        return pltpu.make_async_copy(
            obufs[u], out_hbm.at[b, pl.ds(dst0, BLK * HOP)], osems.at[u])

    def group(idx, carry):
        b = idx // 4
        gi = idx % 4

        @pl.when(gi == 0)
        def _():
            @pl.when(idx > 0)
            def _():
                fetch(b, blk0, 1).wait()
            fetch(b, blk0 - 1, 3).start()
            fetch(b, blk0, 0).start()
            fetch(b, blk0 + 1, 1).start()
            fetch(b, blk0 - 1, 3).wait()
            fetch(b, blk0, 0).wait()

        for u in range(4):
            j = gi * 4 + u
            fetch(b, blk0 + j + 2, (u + 2) % 4).start()
            fetch(b, blk0 + j + 1, (u + 1) % 4).wait()

            @pl.when(idx > 0)
            def _(u=u, b=b, j=j):
                out_copy(b, j, u).wait()

            obuf = obufs[u]
            o0 = row0 + j * BLK

            @plsc.parallel_loop(0, BLK)
            def row(t):
                o = o0 + t
                for half in range(2):
                    h = o + 1 + half
                    cnt = jnp.minimum(jnp.minimum(h + 1, 4), 4099 - h)
                    recip = jnp.where(
                        cnt == 4, jnp.float32(0.25),
                        jnp.where(cnt == 3, jnp.float32(1.0 / 3.0),
                                  jnp.where(cnt == 2, jnp.float32(0.5),
                                            jnp.float32(1.0))))
                    rv = jnp.broadcast_to(recip, (16,))
                    rows = []
                    for k in range(4):
                        gr = h - k
                        valid = jnp.logical_and(gr >= 0, gr < FV)
                        rows.append(jnp.where(valid, gr & 31, ZROW))
                    for c in range(HOP // 32):
                        off = (1 - half) * 256 + 16 * c
                        v = xbuf[rows[0], pl.ds(off, 16)]
                        v = v + xbuf[rows[1], pl.ds(512 + off, 16)]
                        v = v + xbuf[rows[2], pl.ds(1024 + off, 16)]
                        v = v + xbuf[rows[3], pl.ds(1536 + off, 16)]
                        obuf[pl.ds(t * HOP + half * 256 + 16 * c, 16)] = v * rv

            out_copy(b, j, u).start()
        return carry

    lax.fori_loop(0, BV * 4, group, 0)

    fetch(BV - 1, blk0, 1).wait()
    for u in range(4):
        out_copy(BV - 1, 12 + u, u).wait()


def kernel(x):
    mesh = plsc.VectorSubcoreMesh(core_axis_name="c", subcore_axis_name="s")
    f = functools.partial(
        pl.kernel,
        mesh=mesh,
        out_type=jax.ShapeDtypeStruct((BV, OUT_LEN), jnp.float32),
        scratch_types=[
            pltpu.VMEM((ZROW + 1, FRAME), jnp.float32),
            pltpu.VMEM((BLK * HOP,), jnp.float32),
            pltpu.VMEM((BLK * HOP,), jnp.float32),
            pltpu.VMEM((BLK * HOP,), jnp.float32),
            pltpu.VMEM((BLK * HOP,), jnp.float32),
            pltpu.SemaphoreType.DMA((4,)),
            pltpu.SemaphoreType.DMA((4,)),
        ],
    )(_sc_body)
    return f(x)
